# baseline (speedup 1.0000x reference)
import functools

import jax
import jax.numpy as jnp
from jax import lax
from jax.experimental import pallas as pl
from jax.experimental.pallas import tpu as pltpu
from jax.experimental.pallas import tpu_sc as plsc

G = 64
GB = 8
NS = 16
NC = 2
NA = 10240
ZR = 16


def _sc_agg_kernel(seq, n, ep, d):
    nblk = ep // (G * GB)
    nblk2 = nblk // NC
    bpt = nblk2 // NS
    rpt = NA // NS
    mesh = plsc.VectorSubcoreMesh(core_axis_name="c", subcore_axis_name="s")

    @functools.partial(
        pl.kernel,
        out_type=[
            jax.ShapeDtypeStruct((NC * seq * NA, d), jnp.float32),
            jax.ShapeDtypeStruct((NC * NA, 16), jnp.float32),
        ],
        mesh=mesh,
        compiler_params=pltpu.CompilerParams(use_tc_tiling_on_sc=False),
        scratch_types=[
            pltpu.VMEM((4, G, d), jnp.float32),
            pltpu.VMEM((2, GB, G), jnp.int32),
            pltpu.VMEM((2, GB, G), jnp.int32),
            pltpu.VMEM((G, 16), jnp.float32),
            pltpu.VMEM((ZR, d), jnp.float32),
            pltpu.VMEM((ZR, 16), jnp.float32),
            pltpu.VMEM_SHARED((NA, d), jnp.float32),
            pltpu.VMEM_SHARED((NA, 16), jnp.float32),
            pltpu.SemaphoreType.DMA,
            pltpu.SemaphoreType.DMA,
        ],
    )
    def kern(x_hbm, src_hbm, dst_hbm, out_hbm, deg_hbm,
             rows_v, sidx_v, didx_v, ones_v, zer_v, zdeg_v, agg_s, deg_s,
             sem, sem2):
        c = lax.axis_index("c")
        s = lax.axis_index("s")

        zero16 = jnp.zeros((16,), jnp.float32)
        one16 = jnp.ones((16,), jnp.float32)

        def init_zer(i, _):
            for j in range(d // 16):
                zer_v[i, pl.ds(j * 16, 16)] = zero16
            return 0
        lax.fori_loop(0, ZR, init_zer, 0)

        def init_zdeg(i, _):
            zdeg_v[i, :] = zero16
            return 0
        lax.fori_loop(0, ZR, init_zdeg, 0)

        def init_ones(i, _):
            ones_v[i, :] = one16
            return 0
        lax.fori_loop(0, G, init_ones, 0)

        r0 = s * rpt
        b0 = c * nblk2 + s * bpt

        for t in range(seq):
            xt_hbm = x_hbm.at[pl.ds(t * n, n)]

            for j in range(rpt // ZR):
                pltpu.sync_copy(zer_v, agg_s.at[pl.ds(r0 + j * ZR, ZR)])
                if t == 0:
                    pltpu.sync_copy(zdeg_v,
                                    deg_s.at[pl.ds(r0 + j * ZR, ZR)])
            plsc.subcore_barrier()

            LK = 3
            pltpu.sync_copy(src_hbm.at[b0], sidx_v.at[0])
            pltpu.sync_copy(dst_hbm.at[b0], didx_v.at[0])
            for j in range(LK):
                pltpu.async_copy(xt_hbm.at[sidx_v.at[0, j]], rows_v.at[j],
                                 sem)

            def wait_scat():
                pltpu.make_async_copy(rows_v.at[0],
                                      agg_s.at[didx_v.at[0, 0]],
                                      sem2).wait()

            def pair_body(i, _):
                for k in range(2):
                    cur, nxt = k, (k + 1) % 2
                    b = 2 * i + k

                    @pl.when(b + 1 < bpt)
                    def _():
                        pltpu.sync_copy(src_hbm.at[b0 + b + 1],
                                        sidx_v.at[nxt])
                        pltpu.sync_copy(dst_hbm.at[b0 + b + 1],
                                        didx_v.at[nxt])
                    for j in range(GB):
                        if j >= 1:
                            wait_scat()
                        elif k == 1:
                            wait_scat()
                        else:
                            @pl.when(i >= 1)
                            def _():
                                wait_scat()
                        jf = j + LK
                        if jf < GB:
                            pltpu.async_copy(xt_hbm.at[sidx_v.at[cur, jf]],
                                             rows_v.at[jf % 4], sem)
                        else:
                            @pl.when(b + 1 < bpt)
                            def _():
                                pltpu.async_copy(
                                    xt_hbm.at[sidx_v.at[nxt, jf - GB]],
                                    rows_v.at[jf % 4], sem)
                        pltpu.make_async_copy(xt_hbm.at[sidx_v.at[cur, j]],
                                              rows_v.at[j % 4], sem).wait()
                        pltpu.async_copy(rows_v.at[j % 4],
                                         agg_s.at[didx_v.at[cur, j]], sem2,
                                         add=True)
                        if t == 0:
                            pltpu.sync_copy(ones_v,
                                            deg_s.at[didx_v.at[cur, j]],
                                            add=True)
                return 0
            lax.fori_loop(0, bpt // 2, pair_body, 0)
            wait_scat()
            plsc.subcore_barrier()

            pltpu.sync_copy(agg_s.at[pl.ds(r0, rpt)],
                            out_hbm.at[pl.ds((c * seq + t) * NA + r0, rpt)])
            if t == 0:
                pltpu.sync_copy(deg_s.at[pl.ds(r0, rpt)],
                                deg_hbm.at[pl.ds(c * NA + r0, rpt)])

    return kern


def _tc_body(deg0_ref, deg1_ref, x_ref, a0_ref, a1_ref, wl_ref, wr_ref,
             bl_ref, fcw_ref, fcb_ref, h_ref, y_ref):
    deg = deg0_ref[0, :, 0:1] + deg1_ref[0, :, 0:1]
    inv = 1.0 / jnp.maximum(deg, 1.0)
    hp = lax.Precision.HIGHEST
    f32 = jnp.float32
    am = (a0_ref[0] + a1_ref[0]) * inv
    h = (jnp.dot(am, wl_ref[...], preferred_element_type=f32, precision=hp)
         + jnp.dot(x_ref[0], wr_ref[...], preferred_element_type=f32,
                   precision=hp)
         + bl_ref[...])
    h_ref[0] = h
    y_ref[0] = jnp.sum(h * fcw_ref[...], axis=1, keepdims=True) + fcb_ref[0]


def kernel(input, edge_index, W_l, b_l, W_r, fc_w, fc_b):
    seq, n, d = input.shape
    e = edge_index.shape[1]
    epad = NC * 2 * G * GB * NS
    ep = ((e + epad - 1) // epad) * epad
    ng = ep // G

    xf = input.reshape(seq * n, d)
    src = edge_index[0]
    dst = edge_index[1]
    if ep != e:
        src = jnp.concatenate([src, jnp.zeros((ep - e,), src.dtype)])
        dst = jnp.concatenate([dst, jnp.full((ep - e,), n, dst.dtype)])
    src_g = src.reshape(ng // GB, GB, G)
    dst_g = dst.reshape(ng // GB, GB, G)

    agg_flat, deg = _sc_agg_kernel(seq, n, ep, d)(xf, src_g, dst_g)
    agg4 = agg_flat.reshape(NC, seq, NA, d)
    deg2 = deg.reshape(NC, NA, 16)

    blk = 2000
    h3, y3 = pl.pallas_call(
        _tc_body,
        grid=(seq, n // blk),
        in_specs=[
            pl.BlockSpec((1, blk, 16), lambda t, b: (0, b, 0)),
            pl.BlockSpec((1, blk, 16), lambda t, b: (1, b, 0)),
            pl.BlockSpec((1, blk, d), lambda t, b: (t, b, 0)),
            pl.BlockSpec((1, blk, d), lambda t, b: (t, b, 0)),
            pl.BlockSpec((1, blk, d), lambda t, b: (t + seq, b, 0)),
            pl.BlockSpec((d, d), lambda t, b: (0, 0)),
            pl.BlockSpec((d, d), lambda t, b: (0, 0)),
            pl.BlockSpec((1, d), lambda t, b: (0, 0)),
            pl.BlockSpec((1, d), lambda t, b: (0, 0)),
            pl.BlockSpec(memory_space=pltpu.SMEM),
        ],
        out_specs=[
            pl.BlockSpec((1, blk, d), lambda t, b: (t, b, 0)),
            pl.BlockSpec((1, blk, 1), lambda t, b: (t, b, 0)),
        ],
        out_shape=[
            jax.ShapeDtypeStruct((seq, n, d), jnp.float32),
            jax.ShapeDtypeStruct((seq, n, 1), jnp.float32),
        ],
    )(deg2, deg2, input, agg4.reshape(NC * seq, NA, d),
      agg4.reshape(NC * seq, NA, d), W_l, W_r,
      b_l.reshape(1, d), fc_w.T, fc_b)

    return h3, y3[..., 0]


# speedup vs baseline: 1.4686x; 1.4686x over previous
import functools

import jax
import jax.numpy as jnp
from jax import lax
from jax.experimental import pallas as pl
from jax.experimental.pallas import tpu as pltpu
from jax.experimental.pallas import tpu_sc as plsc

G = 128
GB = 8
NS = 16
NC = 2
NA = 10240
ZR = 64


def _sc_agg_kernel(seq, n, ep, d):
    dh = d // NC
    nblk = ep // (G * GB)
    bpt = nblk // NS
    rpt = NA // NS
    mesh = plsc.VectorSubcoreMesh(core_axis_name="c", subcore_axis_name="s")

    @functools.partial(
        pl.kernel,
        out_type=[
            jax.ShapeDtypeStruct((NC * seq * NA, dh), jnp.float32),
            jax.ShapeDtypeStruct((NA, 16), jnp.float32),
        ],
        mesh=mesh,
        compiler_params=pltpu.CompilerParams(use_tc_tiling_on_sc=False),
        scratch_types=[
            pltpu.VMEM((8, G, dh), jnp.float32),
            pltpu.VMEM((2, GB, G), jnp.int32),
            pltpu.VMEM((2, GB, G), jnp.int32),
            pltpu.VMEM((G, 16), jnp.float32),
            pltpu.VMEM((ZR, dh), jnp.float32),
            pltpu.VMEM((ZR, 16), jnp.float32),
            pltpu.VMEM_SHARED((NA, dh), jnp.float32),
            pltpu.VMEM_SHARED((NA, 16), jnp.float32),
            pltpu.SemaphoreType.DMA,
            pltpu.SemaphoreType.DMA,
        ],
    )
    def kern(x_hbm, src_hbm, dst_hbm, out_hbm, deg_hbm,
             rows_v, sidx_v, didx_v, ones_v, zer_v, zdeg_v, agg_s, deg_s,
             sem, sem2):
        c = lax.axis_index("c")
        s = lax.axis_index("s")

        zero16 = jnp.zeros((16,), jnp.float32)
        one16 = jnp.ones((16,), jnp.float32)

        def init_zer(i, _):
            for j in range(dh // 16):
                zer_v[i, pl.ds(j * 16, 16)] = zero16
            return 0
        lax.fori_loop(0, ZR, init_zer, 0)

        def init_zdeg(i, _):
            zdeg_v[i, :] = zero16
            return 0
        lax.fori_loop(0, ZR, init_zdeg, 0)

        def init_ones(i, _):
            ones_v[i, :] = one16
            return 0
        lax.fori_loop(0, G, init_ones, 0)

        r0 = s * rpt
        b0 = s * bpt

        for t in range(seq):
            xt_hbm = x_hbm.at[pl.ds((c * seq + t) * n, n)]

            for j in range(rpt // ZR):
                pltpu.sync_copy(zer_v, agg_s.at[pl.ds(r0 + j * ZR, ZR)])
            if t == 0:
                @pl.when(c == 0)
                def _():
                    for j in range(rpt // ZR):
                        pltpu.sync_copy(zdeg_v,
                                        deg_s.at[pl.ds(r0 + j * ZR, ZR)])
            plsc.subcore_barrier()

            LK = 5
            pltpu.sync_copy(src_hbm.at[b0], sidx_v.at[0])
            pltpu.sync_copy(dst_hbm.at[b0], didx_v.at[0])
            for j in range(LK):
                pltpu.async_copy(xt_hbm.at[sidx_v.at[0, j]], rows_v.at[j],
                                 sem)

            def wait_scat():
                pltpu.make_async_copy(rows_v.at[0],
                                      agg_s.at[didx_v.at[0, 0]],
                                      sem2).wait()

            def pair_body(i, _):
                for k in range(2):
                    cur, nxt = k, (k + 1) % 2
                    b = 2 * i + k

                    @pl.when(b + 1 < bpt)
                    def _():
                        pltpu.sync_copy(src_hbm.at[b0 + b + 1],
                                        sidx_v.at[nxt])
                        pltpu.sync_copy(dst_hbm.at[b0 + b + 1],
                                        didx_v.at[nxt])
                    for j in range(GB):
                        if j - (GB - LK) >= 0:
                            wait_scat()
                        elif k == 1:
                            wait_scat()
                        else:
                            @pl.when(i >= 1)
                            def _():
                                wait_scat()
                        jf = j + LK
                        if jf < GB:
                            pltpu.async_copy(xt_hbm.at[sidx_v.at[cur, jf]],
                                             rows_v.at[jf % 8], sem)
                        else:
                            @pl.when(b + 1 < bpt)
                            def _():
                                pltpu.async_copy(
                                    xt_hbm.at[sidx_v.at[nxt, jf - GB]],
                                    rows_v.at[jf % 8], sem)
                        pltpu.make_async_copy(xt_hbm.at[sidx_v.at[cur, j]],
                                              rows_v.at[j % 8], sem).wait()
                        pltpu.async_copy(rows_v.at[j % 8],
                                         agg_s.at[didx_v.at[cur, j]], sem2,
                                         add=True)
                        if t == 0:
                            @pl.when(c == 0)
                            def _():
                                pltpu.sync_copy(ones_v,
                                                deg_s.at[didx_v.at[cur, j]],
                                                add=True)
                return 0
            lax.fori_loop(0, bpt // 2, pair_body, 0)
            for _ in range(GB - LK):
                wait_scat()
            plsc.subcore_barrier()

            pltpu.sync_copy(agg_s.at[pl.ds(r0, rpt)],
                            out_hbm.at[pl.ds((c * seq + t) * NA + r0, rpt)])
            if t == 0:
                @pl.when(c == 0)
                def _():
                    pltpu.sync_copy(deg_s.at[pl.ds(r0, rpt)],
                                    deg_hbm.at[pl.ds(r0, rpt)])

    return kern


def _tc_body(deg_ref, x_ref, alo_ref, ahi_ref, wll_ref, wlh_ref, wr_ref,
             bl_ref, fcw_ref, fcb_ref, h_ref, y_ref):
    inv = 1.0 / jnp.maximum(deg_ref[:, 0:1], 1.0)
    hp = lax.Precision.HIGHEST
    f32 = jnp.float32
    h = (jnp.dot(alo_ref[0] * inv, wll_ref[...], preferred_element_type=f32,
                 precision=hp)
         + jnp.dot(ahi_ref[0] * inv, wlh_ref[...], preferred_element_type=f32,
                   precision=hp)
         + jnp.dot(x_ref[0], wr_ref[...], preferred_element_type=f32,
                   precision=hp)
         + bl_ref[...])
    h_ref[0] = h
    y_ref[0] = jnp.sum(h * fcw_ref[...], axis=1, keepdims=True) + fcb_ref[0]


def kernel(input, edge_index, W_l, b_l, W_r, fc_w, fc_b):
    seq, n, d = input.shape
    e = edge_index.shape[1]
    dh = d // NC
    epad = 2 * G * GB * NS
    ep = ((e + epad - 1) // epad) * epad
    ng = ep // G

    xc = input.reshape(seq * n, NC, dh).transpose(1, 0, 2).reshape(
        NC * seq * n, dh)
    src = edge_index[0]
    dst = edge_index[1]
    if ep != e:
        src = jnp.concatenate([src, jnp.zeros((ep - e,), src.dtype)])
        dst = jnp.concatenate([dst, jnp.full((ep - e,), n, dst.dtype)])
    src_g = src.reshape(ng // GB, GB, G)
    dst_g = dst.reshape(ng // GB, GB, G)

    agg_flat, deg = _sc_agg_kernel(seq, n, ep, d)(xc, src_g, dst_g)
    agg4 = agg_flat.reshape(NC, seq, NA, dh)

    blk = 2000
    h3, y3 = pl.pallas_call(
        _tc_body,
        grid=(seq, n // blk),
        in_specs=[
            pl.BlockSpec((blk, 16), lambda t, b: (b, 0)),
            pl.BlockSpec((1, blk, d), lambda t, b: (t, b, 0)),
            pl.BlockSpec((1, blk, dh), lambda t, b: (t, b, 0)),
            pl.BlockSpec((1, blk, dh), lambda t, b: (t, b, 0)),
            pl.BlockSpec((dh, d), lambda t, b: (0, 0)),
            pl.BlockSpec((dh, d), lambda t, b: (0, 0)),
            pl.BlockSpec((d, d), lambda t, b: (0, 0)),
            pl.BlockSpec((1, d), lambda t, b: (0, 0)),
            pl.BlockSpec((1, d), lambda t, b: (0, 0)),
            pl.BlockSpec(memory_space=pltpu.SMEM),
        ],
        out_specs=[
            pl.BlockSpec((1, blk, d), lambda t, b: (t, b, 0)),
            pl.BlockSpec((1, blk, 1), lambda t, b: (t, b, 0)),
        ],
        out_shape=[
            jax.ShapeDtypeStruct((seq, n, d), jnp.float32),
            jax.ShapeDtypeStruct((seq, n, 1), jnp.float32),
        ],
    )(deg, input, agg4[0], agg4[1], W_l[:dh], W_l[dh:], W_r,
      b_l.reshape(1, d), fc_w.T, fc_b)

    return h3, y3[..., 0]
